# baseline (speedup 1.0000x reference)
import dataclasses

import jax
import jax.numpy as jnp
from jax import lax
from jax.experimental import pallas as pl
from jax.experimental.pallas import tpu as pltpu
from jax.experimental.pallas import tpu_sc as plsc

P = 0.2
KTC = 8
CH_PER_BLOCK = 24


def _tc_kernel(r_ref, xs_ref, cidx_ref, x_ref, o_ref):
    b = pl.program_id(0)
    r0 = r_ref[b, 0]
    r1 = r_ref[b, 1]
    ch_index = jnp.sum((r0 > xs_ref[...]).astype(jnp.int32))
    active = (r1 < P).astype(jnp.float32)
    onehot = (cidx_ref[...] == ch_index).astype(jnp.float32)
    scale = 1.0 - onehot * active
    o_ref[...] = x_ref[...] * scale


def _sc_body(rT_hbm, xs_hbm, z_hbm, x_hbm, o_hbm,
             r_v, xs_v, buf, sem_small, sem_in, sem_out):
    B, C, H, W = x_hbm.shape
    HB = H // 4
    NSC = B - KTC
    UPS = 32 // NSC
    NCH = C // UPS
    NBLK = NCH * 4

    wid = lax.axis_index("s") * 2 + lax.axis_index("c")
    b = KTC + wid // UPS
    bo = wid // UPS
    c0 = (wid % UPS) * NCH

    pltpu.make_async_copy(rT_hbm, r_v, sem_small).start()
    pltpu.make_async_copy(rT_hbm, r_v, sem_small).wait()
    pltpu.make_async_copy(xs_hbm, xs_v, sem_small).start()
    pltpu.make_async_copy(xs_hbm, xs_v, sem_small).wait()

    lane = lax.iota(jnp.int32, 16)
    sel = (lane == b).astype(jnp.float32)
    r0b = jnp.sum(r_v[0, :] * sel, axis=0)
    r1b = jnp.sum(r_v[1, :] * sel, axis=0)
    cnt = jnp.int32(0)
    for k in range(C // 16):
        cnt = cnt + jnp.sum(
            (r0b > xs_v[pl.ds(k * 16, 16)]).astype(jnp.int32), axis=0)
    drop_mine = jnp.logical_and(
        r1b < P, jnp.logical_and(cnt >= c0, cnt < c0 + NCH))

    def start_in(i, j):
        c = c0 + i // 4
        h0 = (i % 4) * HB
        pltpu.make_async_copy(
            x_hbm.at[b, c, pl.ds(h0, HB), :], buf.at[j], sem_in.at[j]
        ).start()

    def wait_in(j):
        pltpu.make_async_copy(
            x_hbm.at[0, 0, pl.ds(0, HB), :], buf.at[j], sem_in.at[j]
        ).wait()

    def start_out(i, j):
        c = c0 + i // 4
        h0 = (i % 4) * HB
        pltpu.make_async_copy(
            buf.at[j], o_hbm.at[bo, c, pl.ds(h0, HB), :], sem_out.at[j]
        ).start()

    def wait_out(i, j):
        c = c0 + i // 4
        h0 = (i % 4) * HB
        pltpu.make_async_copy(
            buf.at[j], o_hbm.at[bo, c, pl.ds(h0, HB), :], sem_out.at[j]
        ).wait()

    start_in(0, 0)
    start_in(1, 1)
    wait_in(0)
    start_out(0, 0)
    start_in(2, 2)
    wait_in(1)
    start_out(1, 1)

    @pl.loop(3, NBLK, step=3)
    def _(g0):
        for j in range(3):
            i = g0 + j
            wait_out(i - 3, j)
            start_in(i, j)
            jp = (j + 2) % 3
            wait_in(jp)
            start_out(i - 1, jp)

    wait_in(2)
    start_out(NBLK - 1, 2)
    wait_out(NBLK - 3, 0)
    wait_out(NBLK - 2, 1)
    wait_out(NBLK - 1, 2)

    @pl.when(drop_mine)
    def _():
        zb = buf.at[0]
        pltpu.make_async_copy(z_hbm, zb, sem_small).start()
        pltpu.make_async_copy(z_hbm, zb, sem_small).wait()
        for hb in range(4):
            pltpu.make_async_copy(
                zb, o_hbm.at[bo, cnt, pl.ds(hb * HB, HB), :],
                sem_out.at[hb % 3],
            ).start()
        for hb in range(4):
            pltpu.make_async_copy(
                zb, o_hbm.at[bo, cnt, pl.ds(hb * HB, HB), :],
                sem_out.at[hb % 3],
            ).wait()


def kernel(tensor, r):
    B, C, H, W = tensor.shape
    HB = H // 4
    xs = jnp.linspace(1.0 / C, 1.0, C).astype(jnp.float32)
    xs2d = xs.reshape(1, C)
    cidx = jnp.arange(C, dtype=jnp.int32).reshape(1, C, 1, 1)
    rT = r.T.astype(jnp.float32)
    zeros = jnp.zeros((HB, W), jnp.float32)

    CB = CH_PER_BLOCK
    tc_out = pl.pallas_call(
        _tc_kernel,
        grid=(KTC, C // CB),
        in_specs=[
            pl.BlockSpec(memory_space=pltpu.SMEM),
            pl.BlockSpec((1, C), lambda b, j: (0, 0)),
            pl.BlockSpec((1, CB, 1, 1), lambda b, j: (0, j, 0, 0)),
            pl.BlockSpec((1, CB, H, W), lambda b, j: (b, j, 0, 0)),
        ],
        out_specs=pl.BlockSpec((1, CB, H, W), lambda b, j: (b, j, 0, 0)),
        out_shape=jax.ShapeDtypeStruct((KTC, C, H, W), jnp.float32),
    )(r, xs2d, cidx, tensor)

    mesh = plsc.VectorSubcoreMesh(core_axis_name="c", subcore_axis_name="s")
    cp = pltpu.CompilerParams()
    if "needs_layout_passes" in pltpu.CompilerParams.__dataclass_fields__:
        cp = dataclasses.replace(cp, needs_layout_passes=False)
    sc_out = pl.kernel(
        _sc_body,
        out_type=jax.ShapeDtypeStruct((B - KTC, C, H, W), jnp.float32),
        mesh=mesh,
        compiler_params=cp,
        scratch_types=[
            pltpu.VMEM((2, 16), jnp.float32),
            pltpu.VMEM((96,), jnp.float32),
            pltpu.VMEM((3, HB, W), jnp.float32),
            pltpu.SemaphoreType.DMA,
            pltpu.SemaphoreType.DMA((3,)),
            pltpu.SemaphoreType.DMA((3,)),
        ],
    )(rT, xs, zeros, tensor)

    return jnp.concatenate([tc_out, sc_out], axis=0)


# speedup vs baseline: 1.8162x; 1.8162x over previous
import dataclasses

import jax
import jax.numpy as jnp
from jax import lax
from jax.experimental import pallas as pl
from jax.experimental.pallas import tpu as pltpu
from jax.experimental.pallas import tpu_sc as plsc

P = 0.2


def _sc_body(rT_hbm, xs_hbm, z_hbm, x_hbm, o_hbm,
             r_v, xs_v, buf, sem_small, sem_in, sem_out):
    B, C, H, W = x_hbm.shape
    HB = 64
    NH = H // HB
    NCH = 48
    NBLK = NCH * NH

    wid = lax.axis_index("s") * 2 + lax.axis_index("c")
    b = wid // 2
    c0 = (wid % 2) * NCH

    pltpu.make_async_copy(rT_hbm, r_v, sem_small).start()
    pltpu.make_async_copy(rT_hbm, r_v, sem_small).wait()
    pltpu.make_async_copy(xs_hbm, xs_v, sem_small).start()
    pltpu.make_async_copy(xs_hbm, xs_v, sem_small).wait()

    lane = lax.iota(jnp.int32, 16)
    sel = (lane == b).astype(jnp.float32)
    r0b = jnp.sum(r_v[0, :] * sel, axis=0)
    r1b = jnp.sum(r_v[1, :] * sel, axis=0)
    cnt = jnp.int32(0)
    for k in range(C // 16):
        cnt = cnt + jnp.sum(
            (r0b > xs_v[pl.ds(k * 16, 16)]).astype(jnp.int32), axis=0)
    drop_mine = jnp.logical_and(
        r1b < P, jnp.logical_and(cnt >= c0, cnt < c0 + NCH))

    def start_in(i, j):
        c = c0 + i // NH
        h0 = (i % NH) * HB
        pltpu.make_async_copy(
            x_hbm.at[b, c, pl.ds(h0, HB), :], buf.at[j], sem_in.at[j]
        ).start()

    def wait_in(j):
        pltpu.make_async_copy(
            x_hbm.at[0, 0, pl.ds(0, HB), :], buf.at[j], sem_in.at[j]
        ).wait()

    def start_out(i, j):
        c = c0 + i // NH
        h0 = (i % NH) * HB
        pltpu.make_async_copy(
            buf.at[j], o_hbm.at[b, c, pl.ds(h0, HB), :], sem_out.at[j]
        ).start()

    def wait_out(i, j):
        c = c0 + i // NH
        h0 = (i % NH) * HB
        pltpu.make_async_copy(
            buf.at[j], o_hbm.at[b, c, pl.ds(h0, HB), :], sem_out.at[j]
        ).wait()

    start_in(0, 0)
    start_in(1, 1)
    wait_in(0)
    start_out(0, 0)
    start_in(2, 2)
    wait_in(1)
    start_out(1, 1)
    start_in(3, 3)

    @pl.loop(4, NBLK, step=4)
    def _(g0):
        for j in range(4):
            i = g0 + j
            wait_out(i - 4, j)
            start_in(i, j)
            jp = (j + 2) % 4
            wait_in(jp)
            start_out(i - 2, jp)

    wait_in((NBLK - 2) % 4)
    start_out(NBLK - 2, (NBLK - 2) % 4)
    wait_in((NBLK - 1) % 4)
    start_out(NBLK - 1, (NBLK - 1) % 4)
    wait_out(NBLK - 4, 0)
    wait_out(NBLK - 3, 1)
    wait_out(NBLK - 2, 2)
    wait_out(NBLK - 1, 3)

    @pl.when(drop_mine)
    def _():
        zb = buf.at[0]
        pltpu.make_async_copy(z_hbm, zb, sem_small).start()
        pltpu.make_async_copy(z_hbm, zb, sem_small).wait()
        for hb in range(NH):
            pltpu.make_async_copy(
                zb, o_hbm.at[b, cnt, pl.ds(hb * HB, HB), :],
                sem_out.at[hb % 4],
            ).start()
        for hb in range(NH):
            pltpu.make_async_copy(
                zb, o_hbm.at[b, cnt, pl.ds(hb * HB, HB), :],
                sem_out.at[hb % 4],
            ).wait()


def kernel(tensor, r):
    B, C, H, W = tensor.shape
    HB = 64
    xs = jnp.linspace(1.0 / C, 1.0, C).astype(jnp.float32)
    rT = r.T.astype(jnp.float32)
    zeros = jnp.zeros((HB, W), jnp.float32)

    mesh = plsc.VectorSubcoreMesh(core_axis_name="c", subcore_axis_name="s")
    cp = pltpu.CompilerParams()
    if "needs_layout_passes" in pltpu.CompilerParams.__dataclass_fields__:
        cp = dataclasses.replace(cp, needs_layout_passes=False)
    run = pl.kernel(
        _sc_body,
        out_type=jax.ShapeDtypeStruct((B, C, H, W), jnp.float32),
        mesh=mesh,
        compiler_params=cp,
        scratch_types=[
            pltpu.VMEM((2, 16), jnp.float32),
            pltpu.VMEM((96,), jnp.float32),
            pltpu.VMEM((4, HB, W), jnp.float32),
            pltpu.SemaphoreType.DMA,
            pltpu.SemaphoreType.DMA((4,)),
            pltpu.SemaphoreType.DMA((4,)),
        ],
    )
    return run(rT, xs, zeros, tensor)


# speedup vs baseline: 2.0320x; 1.1188x over previous
import dataclasses

import jax
import jax.numpy as jnp
from jax import lax
from jax.experimental import pallas as pl
from jax.experimental.pallas import tpu as pltpu
from jax.experimental.pallas import tpu_sc as plsc

P = 0.2
CH_PER_BLOCK = 24


def _copy_kernel(x_ref, o_ref):
    o_ref[...] = x_ref[...]


def _sc_target_body(rT_hbm, xs_hbm, o_hbm, r_v, xs_v, t_v, sem):
    C = xs_hbm.shape[0]
    B = o_hbm.shape[1]
    wid = lax.axis_index("s") * 2 + lax.axis_index("c")

    @pl.when(wid == 0)
    def _():
        pltpu.make_async_copy(rT_hbm, r_v, sem).start()
        pltpu.make_async_copy(rT_hbm, r_v, sem).wait()
        pltpu.make_async_copy(xs_hbm, xs_v, sem).start()
        pltpu.make_async_copy(xs_hbm, xs_v, sem).wait()

        lane = lax.iota(jnp.int32, 16)
        r0 = r_v[0, :]
        r1 = r_v[1, :]
        tgt = jnp.full((16,), -1, jnp.int32)
        for b in range(B):
            self = (lane == b).astype(jnp.float32)
            r0b = jnp.sum(r0 * self, axis=0)
            r1b = jnp.sum(r1 * self, axis=0)
            cnt = jnp.int32(0)
            for k in range(C // 16):
                cnt = cnt + jnp.sum(
                    (r0b > xs_v[pl.ds(k * 16, 16)]).astype(jnp.int32), axis=0)
            tb = jnp.where(r1b < P, cnt, jnp.int32(-1))
            tgt = jnp.where(lane == b, tb, tgt)
        t_v[...] = tgt
        pltpu.make_async_copy(t_v, o_hbm.at[0], sem).start()
        pltpu.make_async_copy(t_v, o_hbm.at[0], sem).wait()


def _scatter_kernel(t_ref, a_ref, o_ref, zbuf, sem):
    B, C, H, W = o_ref.shape
    zbuf[...] = jnp.zeros((H, W), jnp.float32)

    def body(b, _):
        tgt = t_ref[0, b]

        @pl.when(tgt >= 0)
        def _():
            cp = pltpu.make_async_copy(zbuf, o_ref.at[b, tgt], sem)
            cp.start()
            cp.wait()

        return 0

    jax.lax.fori_loop(0, B, body, 0)


def kernel(tensor, r):
    B, C, H, W = tensor.shape
    xs = jnp.linspace(1.0 / C, 1.0, C).astype(jnp.float32)
    rT = r.T.astype(jnp.float32)

    CB = CH_PER_BLOCK
    copied = pl.pallas_call(
        _copy_kernel,
        grid=(B, C // CB),
        in_specs=[pl.BlockSpec((1, CB, H, W), lambda b, j: (b, j, 0, 0))],
        out_specs=pl.BlockSpec((1, CB, H, W), lambda b, j: (b, j, 0, 0)),
        out_shape=jax.ShapeDtypeStruct((B, C, H, W), jnp.float32),
    )(tensor)

    mesh = plsc.VectorSubcoreMesh(core_axis_name="c", subcore_axis_name="s")
    cp = pltpu.CompilerParams()
    if "needs_layout_passes" in pltpu.CompilerParams.__dataclass_fields__:
        cp = dataclasses.replace(cp, needs_layout_passes=False)
    targets = pl.kernel(
        _sc_target_body,
        out_type=jax.ShapeDtypeStruct((1, B), jnp.int32),
        mesh=mesh,
        compiler_params=cp,
        scratch_types=[
            pltpu.VMEM((2, 16), jnp.float32),
            pltpu.VMEM((C,), jnp.float32),
            pltpu.VMEM((16,), jnp.int32),
            pltpu.SemaphoreType.DMA,
        ],
    )(rT, xs)

    out = pl.pallas_call(
        _scatter_kernel,
        in_specs=[
            pl.BlockSpec(memory_space=pltpu.SMEM),
            pl.BlockSpec(memory_space=pltpu.MemorySpace.HBM),
        ],
        out_specs=pl.BlockSpec(memory_space=pltpu.MemorySpace.HBM),
        out_shape=jax.ShapeDtypeStruct((B, C, H, W), jnp.float32),
        scratch_shapes=[
            pltpu.VMEM((H, W), jnp.float32),
            pltpu.SemaphoreType.DMA,
        ],
        input_output_aliases={1: 0},
    )(targets, copied)
    return out
